# baseline (speedup 1.0000x reference)
import jax
import jax.numpy as jnp
from jax.experimental import pallas as pl

_MBLK = 256


def _add_body(x_ref, p_ref, o_ref):
    o_ref[...] = x_ref[...] + p_ref[...][None, :, :]


def kernel(x, pos_table):
    B, M, D = x.shape
    grid = (M // _MBLK,)
    return pl.pallas_call(
        _add_body,
        grid=grid,
        in_specs=[
            pl.BlockSpec((B, _MBLK, D), lambda i: (0, i, 0)),
            pl.BlockSpec((_MBLK, D), lambda i: (i, 0)),
        ],
        out_specs=pl.BlockSpec((B, _MBLK, D), lambda i: (0, i, 0)),
        out_shape=jax.ShapeDtypeStruct((B, M, D), x.dtype),
    )(x, pos_table)
